# baseline (speedup 1.0000x reference)
import functools

import jax
import jax.numpy as jnp
from jax import lax
from jax.experimental import pallas as pl
from jax.experimental.pallas import tpu as pltpu
from jax.experimental.pallas import tpu_sc as plsc

N_NODES = 10000
N_EDGES = 320000
D = 128

NC = 2
NS = 16
NW = NC * NS
E_W = N_EDGES // NW
C = 128
K = 80
E_WP = K * C
NE_P = NW * E_WP
NP = 10240
RPT = NP // NS

CH = 200
KH = E_W // CH

_mesh = plsc.VectorSubcoreMesh(core_axis_name="c", subcore_axis_name="s")


def _fill(buf, rows, width, value):
    @pl.loop(0, rows)
    def _(r):
        for c in range(width // 16):
            buf.at[pl.ds(r, 1), pl.ds(c * 16, 16)][...] = jnp.full(
                (1, 16), value, jnp.float32)


def _zero_table(table, zbuf, zrows, base_r, rows):
    @pl.loop(0, rows // zrows)
    def _(i):
        pltpu.sync_copy(zbuf, table.at[pl.ds(base_r + i * zrows, zrows)])


@functools.partial(
    pl.kernel,
    out_type=jax.ShapeDtypeStruct((NC, NP, 16), jnp.float32),
    mesh=_mesh,
    scratch_types=[
        pltpu.VMEM((CH,), jnp.int32),
        pltpu.VMEM((CH, 16), jnp.float32),
        pltpu.VMEM((64, 16), jnp.float32),
        pltpu.VMEM_SHARED((NP, 16), jnp.float32),
    ],
)
def _sc_hist(dst_hbm, out_hbm, dst_v, ones_v, zbuf, table):
    cid = lax.axis_index("c")
    sid = lax.axis_index("s")
    wid = sid * NC + cid

    _fill(zbuf, 64, 16, 0.0)
    _fill(ones_v, CH, 16, 1.0)
    base_r = sid * RPT
    _zero_table(table, zbuf, 64, base_r, RPT)
    plsc.subcore_barrier()

    base_e = wid * E_W

    @pl.loop(0, KH)
    def _(j):
        pltpu.sync_copy(dst_hbm.at[pl.ds(base_e + j * CH, CH)], dst_v)
        pltpu.sync_copy(ones_v, table.at[dst_v], add=True)

    plsc.subcore_barrier()
    pltpu.sync_copy(table.at[pl.ds(base_r, RPT)],
                    out_hbm.at[cid, pl.ds(base_r, RPT)])


@functools.partial(
    pl.kernel,
    out_type=jax.ShapeDtypeStruct((NC, NP, D), jnp.float32),
    mesh=_mesh,
    scratch_types=[
        pltpu.VMEM((C,), jnp.int32),
        pltpu.VMEM((C,), jnp.int32),
        pltpu.VMEM((C,), jnp.int32),
        pltpu.VMEM((C,), jnp.int32),
        pltpu.VMEM((C, D), jnp.float32),
        pltpu.VMEM((C, D), jnp.float32),
        pltpu.VMEM((32, D), jnp.float32),
        pltpu.VMEM_SHARED((NP, D), jnp.float32),
        pltpu.SemaphoreType.DMA,
        pltpu.SemaphoreType.DMA,
    ],
)
def _sc_scatter(hs_hbm, src_hbm, dst_hbm, out_hbm, src_0, src_1, dst_0,
                dst_1, rows_0, rows_1, zbuf, table, sem_0, sem_1):
    cid = lax.axis_index("c")
    sid = lax.axis_index("s")
    wid = sid * NC + cid
    src_v = (src_0, src_1)
    dst_v = (dst_0, dst_1)
    rows_v = (rows_0, rows_1)
    sems = (sem_0, sem_1)

    _fill(zbuf, 32, D, 0.0)
    base_r = sid * RPT
    _zero_table(table, zbuf, 32, base_r, RPT)
    plsc.subcore_barrier()

    base_e = wid * E_WP

    @pl.loop(0, K)
    def _(j):
        off = base_e + j * C
        pltpu.sync_copy(src_hbm.at[pl.ds(off, C)], src_v[0])
        pltpu.sync_copy(dst_hbm.at[pl.ds(off, C)], dst_v[0])
        pltpu.async_copy(hs_hbm.at[src_v[0]], rows_v[0], sems[0]).wait()
        pltpu.sync_copy(rows_v[0], table.at[dst_v[0]], add=True)

    plsc.subcore_barrier()
    pltpu.sync_copy(table.at[pl.ds(base_r, RPT)],
                    out_hbm.at[cid, pl.ds(base_r, RPT)])


BLK = 2000
_GRID = N_NODES // BLK


def _row_spec(w):
    return pl.BlockSpec((BLK, w), lambda i: (i, 0))


def _full_spec(a, b):
    return pl.BlockSpec((a, b), lambda i: (0, 0))


def _mm_body(x_ref, w_ref, o_ref):
    o_ref[...] = jnp.dot(x_ref[...], w_ref[...],
                         preferred_element_type=jnp.float32)


def _mm(x, w):
    return pl.pallas_call(
        _mm_body,
        grid=(_GRID,),
        in_specs=[_row_spec(D), _full_spec(D, D)],
        out_specs=_row_spec(D),
        out_shape=jax.ShapeDtypeStruct((N_NODES, D), jnp.float32),
    )(x, w)


def _dinv(d0_ref, d1_ref):
    deg = d0_ref[...][:, :1] + d1_ref[...][:, :1] + 1.0
    return lax.rsqrt(deg)


def _scale_body(h_ref, d0_ref, d1_ref, o_ref):
    o_ref[...] = h_ref[...] * _dinv(d0_ref, d1_ref)


def _scale(h, d0, d1):
    return pl.pallas_call(
        _scale_body,
        grid=(_GRID,),
        in_specs=[_row_spec(D), _row_spec(16), _row_spec(16)],
        out_specs=_row_spec(D),
        out_shape=jax.ShapeDtypeStruct((N_NODES, D), jnp.float32),
    )(h, d0, d1)


def _mid_body(p0_ref, p1_ref, hs_ref, d0_ref, d1_ref, b_ref, w_ref, o_ref):
    dinv = _dinv(d0_ref, d1_ref)
    agg = p0_ref[...] + p1_ref[...]
    h = jnp.maximum(dinv * (agg + hs_ref[...]) + b_ref[...], 0.0)
    o_ref[...] = jnp.dot(h, w_ref[...],
                         preferred_element_type=jnp.float32) * dinv


def _mid(p0, p1, hs, d0, d1, b, w):
    return pl.pallas_call(
        _mid_body,
        grid=(_GRID,),
        in_specs=[_row_spec(D), _row_spec(D), _row_spec(D), _row_spec(16),
                  _row_spec(16), _full_spec(1, D), _full_spec(D, D)],
        out_specs=_row_spec(D),
        out_shape=jax.ShapeDtypeStruct((N_NODES, D), jnp.float32),
    )(p0, p1, hs, d0, d1, b, w)


def _out_body(p0_ref, p1_ref, hs_ref, d0_ref, d1_ref, b_ref, w_ref, b3_ref,
              o_ref):
    dinv = _dinv(d0_ref, d1_ref)
    agg = p0_ref[...] + p1_ref[...]
    h = jnp.maximum(dinv * (agg + hs_ref[...]) + b_ref[...], 0.0)
    o_ref[...] = jnp.dot(h, w_ref[...],
                         preferred_element_type=jnp.float32) + b3_ref[...]


def _out(p0, p1, hs, d0, d1, b, w3p, b3p):
    return pl.pallas_call(
        _out_body,
        grid=(_GRID,),
        in_specs=[_row_spec(D), _row_spec(D), _row_spec(D), _row_spec(16),
                  _row_spec(16), _full_spec(1, D), _full_spec(D, D),
                  _full_spec(1, D)],
        out_specs=_row_spec(D),
        out_shape=jax.ShapeDtypeStruct((N_NODES, D), jnp.float32),
    )(p0, p1, hs, d0, d1, b, w3p, b3p)


def kernel(x, edge_index, W1, b1, W2, b2, W3, b3):
    src = edge_index[0].astype(jnp.int32)
    dst = edge_index[1].astype(jnp.int32)
    pad = NE_P - N_EDGES
    src_p = jnp.concatenate([src, jnp.zeros((pad,), jnp.int32)])
    pad_dst = N_NODES + jnp.arange(pad, dtype=jnp.int32) % (NP - N_NODES)
    dst_p = jnp.concatenate([dst, pad_dst])

    out_ch = W3.shape[1]
    w3p = jnp.zeros((D, D), jnp.float32).at[:, :out_ch].set(W3)
    b3p = jnp.zeros((1, D), jnp.float32).at[:, :out_ch].set(b3)

    deg_p = _sc_hist(dst)
    h_raw1 = _mm(x, W1)
    d0 = deg_p[0, :N_NODES]
    d1 = deg_p[1, :N_NODES]

    h1s = _scale(h_raw1, d0, d1)
    p1 = _sc_scatter(h1s, src_p, dst_p)
    h2s = _mid(p1[0, :N_NODES], p1[1, :N_NODES], h1s, d0, d1,
               b1.reshape(1, D), W2)
    p2 = _sc_scatter(h2s, src_p, dst_p)
    outp = _out(p2[0, :N_NODES], p2[1, :N_NODES], h2s, d0, d1,
                b2.reshape(1, D), w3p, b3p)
    return outp[:, :out_ch]


# speedup vs baseline: 1.1412x; 1.1412x over previous
import functools

import jax
import jax.numpy as jnp
from jax import lax
from jax.experimental import pallas as pl
from jax.experimental.pallas import tpu as pltpu
from jax.experimental.pallas import tpu_sc as plsc

N_NODES = 10000
N_EDGES = 320000
D = 128

NC = 2
NS = 16
NW = NC * NS
E_W = N_EDGES // NW
C = 160
K = 64
E_WP = K * C
NE_P = NW * E_WP
NP = 10240
RPT = NP // NS

CH = 200
KH = E_W // CH

_mesh = plsc.VectorSubcoreMesh(core_axis_name="c", subcore_axis_name="s")


def _fill(buf, rows, width, value):
    @pl.loop(0, rows)
    def _(r):
        for c in range(width // 16):
            buf.at[pl.ds(r, 1), pl.ds(c * 16, 16)][...] = jnp.full(
                (1, 16), value, jnp.float32)


def _zero_table(table, zbuf, zrows, base_r, rows):
    @pl.loop(0, rows // zrows)
    def _(i):
        pltpu.sync_copy(zbuf, table.at[pl.ds(base_r + i * zrows, zrows)])


@functools.partial(
    pl.kernel,
    out_type=jax.ShapeDtypeStruct((NC, NP, 16), jnp.float32),
    mesh=_mesh,
    scratch_types=[
        pltpu.VMEM((CH,), jnp.int32),
        pltpu.VMEM((CH, 16), jnp.float32),
        pltpu.VMEM((64, 16), jnp.float32),
        pltpu.VMEM_SHARED((NP, 16), jnp.float32),
    ],
)
def _sc_hist(dst_hbm, out_hbm, dst_v, ones_v, zbuf, table):
    cid = lax.axis_index("c")
    sid = lax.axis_index("s")
    wid = sid * NC + cid

    _fill(zbuf, 64, 16, 0.0)
    _fill(ones_v, CH, 16, 1.0)
    base_r = sid * RPT
    _zero_table(table, zbuf, 64, base_r, RPT)
    plsc.subcore_barrier()

    base_e = wid * E_W

    @pl.loop(0, KH)
    def _(j):
        pltpu.sync_copy(dst_hbm.at[pl.ds(base_e + j * CH, CH)], dst_v)
        pltpu.sync_copy(ones_v, table.at[dst_v], add=True)

    plsc.subcore_barrier()
    pltpu.sync_copy(table.at[pl.ds(base_r, RPT)],
                    out_hbm.at[cid, pl.ds(base_r, RPT)])


@functools.partial(
    pl.kernel,
    out_type=jax.ShapeDtypeStruct((NC, NP, D), jnp.float32),
    mesh=_mesh,
    scratch_types=[
        pltpu.VMEM((C,), jnp.int32),
        pltpu.VMEM((C,), jnp.int32),
        pltpu.VMEM((C,), jnp.int32),
        pltpu.VMEM((C,), jnp.int32),
        pltpu.VMEM((C, D), jnp.float32),
        pltpu.VMEM((C, D), jnp.float32),
        pltpu.VMEM((32, D), jnp.float32),
        pltpu.VMEM_SHARED((NP, D), jnp.float32),
        pltpu.SemaphoreType.DMA,
        pltpu.SemaphoreType.DMA,
    ],
)
def _sc_scatter(hs_hbm, src_hbm, dst_hbm, out_hbm, src_0, src_1, dst_0,
                dst_1, rows_0, rows_1, zbuf, table, sem_0, sem_1):
    cid = lax.axis_index("c")
    sid = lax.axis_index("s")
    wid = sid * NC + cid
    src_v = (src_0, src_1)
    dst_v = (dst_0, dst_1)
    rows_v = (rows_0, rows_1)
    sems = (sem_0, sem_1)

    _fill(zbuf, 32, D, 0.0)
    base_r = sid * RPT
    _zero_table(table, zbuf, 32, base_r, RPT)
    plsc.subcore_barrier()

    base_e = wid * E_WP

    @pl.loop(0, K // 2)
    def _(jj):
        j = jj * 2
        descs = []
        for b in range(2):
            off = base_e + (j + b) * C
            pltpu.sync_copy(src_hbm.at[pl.ds(off, C)], src_v[b])
            pltpu.sync_copy(dst_hbm.at[pl.ds(off, C)], dst_v[b])
            descs.append(
                pltpu.async_copy(hs_hbm.at[src_v[b]], rows_v[b], sems[b]))
        for b in range(2):
            descs[b].wait()
            pltpu.sync_copy(rows_v[b], table.at[dst_v[b]], add=True)

    plsc.subcore_barrier()
    pltpu.sync_copy(table.at[pl.ds(base_r, RPT)],
                    out_hbm.at[cid, pl.ds(base_r, RPT)])


BLK = 2000
_GRID = N_NODES // BLK


def _row_spec(w):
    return pl.BlockSpec((BLK, w), lambda i: (i, 0))


def _full_spec(a, b):
    return pl.BlockSpec((a, b), lambda i: (0, 0))


def _mm_body(x_ref, w_ref, o_ref):
    o_ref[...] = jnp.dot(x_ref[...], w_ref[...],
                         preferred_element_type=jnp.float32)


def _mm(x, w):
    return pl.pallas_call(
        _mm_body,
        grid=(_GRID,),
        in_specs=[_row_spec(D), _full_spec(D, D)],
        out_specs=_row_spec(D),
        out_shape=jax.ShapeDtypeStruct((N_NODES, D), jnp.float32),
    )(x, w)


def _dinv(d0_ref, d1_ref):
    deg = d0_ref[...][:, :1] + d1_ref[...][:, :1] + 1.0
    return lax.rsqrt(deg)


def _scale_body(h_ref, d0_ref, d1_ref, o_ref):
    o_ref[...] = h_ref[...] * _dinv(d0_ref, d1_ref)


def _scale(h, d0, d1):
    return pl.pallas_call(
        _scale_body,
        grid=(_GRID,),
        in_specs=[_row_spec(D), _row_spec(16), _row_spec(16)],
        out_specs=_row_spec(D),
        out_shape=jax.ShapeDtypeStruct((N_NODES, D), jnp.float32),
    )(h, d0, d1)


def _mid_body(p0_ref, p1_ref, hs_ref, d0_ref, d1_ref, b_ref, w_ref, o_ref):
    dinv = _dinv(d0_ref, d1_ref)
    agg = p0_ref[...] + p1_ref[...]
    h = jnp.maximum(dinv * (agg + hs_ref[...]) + b_ref[...], 0.0)
    o_ref[...] = jnp.dot(h, w_ref[...],
                         preferred_element_type=jnp.float32) * dinv


def _mid(p0, p1, hs, d0, d1, b, w):
    return pl.pallas_call(
        _mid_body,
        grid=(_GRID,),
        in_specs=[_row_spec(D), _row_spec(D), _row_spec(D), _row_spec(16),
                  _row_spec(16), _full_spec(1, D), _full_spec(D, D)],
        out_specs=_row_spec(D),
        out_shape=jax.ShapeDtypeStruct((N_NODES, D), jnp.float32),
    )(p0, p1, hs, d0, d1, b, w)


def _out_body(p0_ref, p1_ref, hs_ref, d0_ref, d1_ref, b_ref, w_ref, b3_ref,
              o_ref):
    dinv = _dinv(d0_ref, d1_ref)
    agg = p0_ref[...] + p1_ref[...]
    h = jnp.maximum(dinv * (agg + hs_ref[...]) + b_ref[...], 0.0)
    o_ref[...] = jnp.dot(h, w_ref[...],
                         preferred_element_type=jnp.float32) + b3_ref[...]


def _out(p0, p1, hs, d0, d1, b, w3p, b3p):
    return pl.pallas_call(
        _out_body,
        grid=(_GRID,),
        in_specs=[_row_spec(D), _row_spec(D), _row_spec(D), _row_spec(16),
                  _row_spec(16), _full_spec(1, D), _full_spec(D, D),
                  _full_spec(1, D)],
        out_specs=_row_spec(D),
        out_shape=jax.ShapeDtypeStruct((N_NODES, D), jnp.float32),
    )(p0, p1, hs, d0, d1, b, w3p, b3p)


def kernel(x, edge_index, W1, b1, W2, b2, W3, b3):
    src = edge_index[0].astype(jnp.int32)
    dst = edge_index[1].astype(jnp.int32)
    pad = NE_P - N_EDGES
    src_p = jnp.concatenate([src, jnp.zeros((pad,), jnp.int32)])
    pad_dst = N_NODES + jnp.arange(pad, dtype=jnp.int32) % (NP - N_NODES)
    dst_p = jnp.concatenate([dst, pad_dst])

    out_ch = W3.shape[1]
    w3p = jnp.zeros((D, D), jnp.float32).at[:, :out_ch].set(W3)
    b3p = jnp.zeros((1, D), jnp.float32).at[:, :out_ch].set(b3)

    deg_p = _sc_hist(dst)
    h_raw1 = _mm(x, W1)
    d0 = deg_p[0, :N_NODES]
    d1 = deg_p[1, :N_NODES]

    h1s = _scale(h_raw1, d0, d1)
    p1 = _sc_scatter(h1s, src_p, dst_p)
    h2s = _mid(p1[0, :N_NODES], p1[1, :N_NODES], h1s, d0, d1,
               b1.reshape(1, D), W2)
    p2 = _sc_scatter(h2s, src_p, dst_p)
    outp = _out(p2[0, :N_NODES], p2[1, :N_NODES], h2s, d0, d1,
                b2.reshape(1, D), w3p, b3p)
    return outp[:, :out_ch]


# speedup vs baseline: 2.7428x; 2.4035x over previous
import functools

import jax
import jax.numpy as jnp
from jax import lax
from jax.experimental import pallas as pl
from jax.experimental.pallas import tpu as pltpu
from jax.experimental.pallas import tpu_sc as plsc

N_NODES = 10000
N_EDGES = 320000
D = 128

NC = 2
NS = 16
NW = NC * NS
E_W = N_EDGES // NW
C = 160
K = 64
E_WP = K * C
NE_P = NW * E_WP
NP = 10240
RPT = NP // NS

CH = 200
KH = E_W // CH

_mesh = plsc.VectorSubcoreMesh(core_axis_name="c", subcore_axis_name="s")


def _fill(buf, rows, width, value):
    @pl.loop(0, rows)
    def _(r):
        for c in range(width // 16):
            buf.at[pl.ds(r, 1), pl.ds(c * 16, 16)][...] = jnp.full(
                (1, 16), value, jnp.float32)


def _zero_table(table, zbuf, zrows, base_r, rows):
    @pl.loop(0, rows // zrows)
    def _(i):
        pltpu.sync_copy(zbuf, table.at[pl.ds(base_r + i * zrows, zrows)])


@functools.partial(
    pl.kernel,
    out_type=jax.ShapeDtypeStruct((NC, NP, 16), jnp.float32),
    mesh=_mesh,
    scratch_types=[
        pltpu.VMEM((CH,), jnp.int32),
        pltpu.VMEM((CH, 16), jnp.float32),
        pltpu.VMEM((64, 16), jnp.float32),
        pltpu.VMEM_SHARED((NP, 16), jnp.float32),
    ],
)
def _sc_hist(dst_hbm, out_hbm, dst_v, ones_v, zbuf, table):
    cid = lax.axis_index("c")
    sid = lax.axis_index("s")
    wid = sid * NC + cid

    _fill(zbuf, 64, 16, 0.0)
    _fill(ones_v, CH, 16, 1.0)
    base_r = sid * RPT
    _zero_table(table, zbuf, 64, base_r, RPT)
    plsc.subcore_barrier()

    base_e = wid * E_W

    @pl.loop(0, KH)
    def _(j):
        pltpu.sync_copy(dst_hbm.at[pl.ds(base_e + j * CH, CH)], dst_v)
        pltpu.sync_copy(ones_v, table.at[dst_v], add=True)

    plsc.subcore_barrier()
    pltpu.sync_copy(table.at[pl.ds(base_r, RPT)],
                    out_hbm.at[cid, pl.ds(base_r, RPT)])


@functools.partial(
    pl.kernel,
    out_type=jax.ShapeDtypeStruct((NC, NP, D), jnp.float32),
    mesh=_mesh,
    scratch_types=[
        pltpu.VMEM((C,), jnp.int32),
        pltpu.VMEM((C,), jnp.int32),
        pltpu.VMEM((C,), jnp.int32),
        pltpu.VMEM((C,), jnp.int32),
        pltpu.VMEM((C, D), jnp.float32),
        pltpu.VMEM((C, D), jnp.float32),
        pltpu.VMEM((32, D), jnp.float32),
        pltpu.VMEM_SHARED((NP, D), jnp.float32),
        pltpu.SemaphoreType.DMA,
        pltpu.SemaphoreType.DMA,
    ],
)
def _sc_scatter(hs_hbm, src_hbm, dst_hbm, out_hbm, src_0, src_1, dst_0,
                dst_1, rows_0, rows_1, zbuf, table, sem_0, sem_1):
    cid = lax.axis_index("c")
    sid = lax.axis_index("s")
    wid = sid * NC + cid
    src_v = (src_0, src_1)
    dst_v = (dst_0, dst_1)
    rows_v = (rows_0, rows_1)
    sems = (sem_0, sem_1)

    _fill(zbuf, 32, D, 0.0)
    base_r = sid * RPT
    _zero_table(table, zbuf, 32, base_r, RPT)
    plsc.subcore_barrier()

    base_e = wid * E_WP

    @pl.loop(0, K // 2)
    def _(jj):
        j = jj * 2
        descs = []
        for b in range(2):
            off = base_e + (j + b) * C
            pltpu.sync_copy(src_hbm.at[pl.ds(off, C)], src_v[b])
            pltpu.sync_copy(dst_hbm.at[pl.ds(off, C)], dst_v[b])
            descs.append(
                pltpu.async_copy(hs_hbm.at[src_v[b]], rows_v[b], sems[b]))
        for b in range(2):
            descs[b].wait()
            pltpu.sync_copy(rows_v[b], table.at[dst_v[b]], add=True)

    plsc.subcore_barrier()
    pltpu.sync_copy(table.at[pl.ds(base_r, RPT)],
                    out_hbm.at[cid, pl.ds(base_r, RPT)])


BLK = 2000
_GRID = N_NODES // BLK


def _row_spec(w):
    return pl.BlockSpec((BLK, w), lambda i: (i, 0))


def _full_spec(a, b):
    return pl.BlockSpec((a, b), lambda i: (0, 0))


def _mm_body(x_ref, w_ref, o_ref):
    o_ref[...] = jnp.dot(x_ref[...], w_ref[...],
                         preferred_element_type=jnp.float32)


def _mm(x, w):
    return pl.pallas_call(
        _mm_body,
        grid=(_GRID,),
        in_specs=[_row_spec(D), _full_spec(D, D)],
        out_specs=_row_spec(D),
        out_shape=jax.ShapeDtypeStruct((N_NODES, D), jnp.float32),
    )(x, w)


def _dinv(d0_ref, d1_ref):
    deg = d0_ref[...][:, :1] + d1_ref[...][:, :1] + 1.0
    return lax.rsqrt(deg)


def _scale_body(h_ref, d0_ref, d1_ref, o_ref):
    o_ref[...] = h_ref[...] * _dinv(d0_ref, d1_ref)


def _scale(h, d0, d1):
    return pl.pallas_call(
        _scale_body,
        grid=(_GRID,),
        in_specs=[_row_spec(D), _row_spec(16), _row_spec(16)],
        out_specs=_row_spec(D),
        out_shape=jax.ShapeDtypeStruct((N_NODES, D), jnp.float32),
    )(h, d0, d1)


def _mid_body(p0_ref, p1_ref, hs_ref, d0_ref, d1_ref, b_ref, w_ref, o_ref):
    dinv = _dinv(d0_ref, d1_ref)
    agg = p0_ref[...] + p1_ref[...]
    h = jnp.maximum(dinv * (agg + hs_ref[...]) + b_ref[...], 0.0)
    o_ref[...] = jnp.dot(h, w_ref[...],
                         preferred_element_type=jnp.float32) * dinv


def _mid(p0, p1, hs, d0, d1, b, w):
    return pl.pallas_call(
        _mid_body,
        grid=(_GRID,),
        in_specs=[_row_spec(D), _row_spec(D), _row_spec(D), _row_spec(16),
                  _row_spec(16), _full_spec(1, D), _full_spec(D, D)],
        out_specs=_row_spec(D),
        out_shape=jax.ShapeDtypeStruct((N_NODES, D), jnp.float32),
    )(p0, p1, hs, d0, d1, b, w)


def _out_body(p0_ref, p1_ref, hs_ref, d0_ref, d1_ref, b_ref, w_ref, b3_ref,
              o_ref):
    dinv = _dinv(d0_ref, d1_ref)
    agg = p0_ref[...] + p1_ref[...]
    h = jnp.maximum(dinv * (agg + hs_ref[...]) + b_ref[...], 0.0)
    o_ref[...] = jnp.dot(h, w_ref[...],
                         preferred_element_type=jnp.float32) + b3_ref[...]


def _out(p0, p1, hs, d0, d1, b, w3p, b3p):
    return pl.pallas_call(
        _out_body,
        grid=(_GRID,),
        in_specs=[_row_spec(D), _row_spec(D), _row_spec(D), _row_spec(16),
                  _row_spec(16), _full_spec(1, D), _full_spec(D, D),
                  _full_spec(1, D)],
        out_specs=_row_spec(D),
        out_shape=jax.ShapeDtypeStruct((N_NODES, D), jnp.float32),
    )(p0, p1, hs, d0, d1, b, w3p, b3p)


def kernel(x, edge_index, W1, b1, W2, b2, W3, b3):
    src = edge_index[0].astype(jnp.int32)
    dst = edge_index[1].astype(jnp.int32)
    pad_w = E_WP - E_W
    pad_src = jnp.broadcast_to(
        jnp.arange(pad_w, dtype=jnp.int32) % N_NODES, (NW, pad_w))
    pad_dst = jnp.broadcast_to(
        N_NODES + jnp.arange(pad_w, dtype=jnp.int32) % (NP - N_NODES),
        (NW, pad_w))
    src_p = jnp.concatenate([src.reshape(NW, E_W), pad_src], 1).reshape(-1)
    dst_p = jnp.concatenate([dst.reshape(NW, E_W), pad_dst], 1).reshape(-1)

    out_ch = W3.shape[1]
    w3p = jnp.zeros((D, D), jnp.float32).at[:, :out_ch].set(W3)
    b3p = jnp.zeros((1, D), jnp.float32).at[:, :out_ch].set(b3)

    deg_p = _sc_hist(dst)
    h_raw1 = _mm(x, W1)
    d0 = deg_p[0, :N_NODES]
    d1 = deg_p[1, :N_NODES]

    h1s = _scale(h_raw1, d0, d1)
    p1 = _sc_scatter(h1s, src_p, dst_p)
    h2s = _mid(p1[0, :N_NODES], p1[1, :N_NODES], h1s, d0, d1,
               b1.reshape(1, D), W2)
    p2 = _sc_scatter(h2s, src_p, dst_p)
    outp = _out(p2[0, :N_NODES], p2[1, :N_NODES], h2s, d0, d1,
                b2.reshape(1, D), w3p, b3p)
    return outp[:, :out_ch]


# speedup vs baseline: 3.3199x; 1.2104x over previous
import functools

import jax
import jax.numpy as jnp
from jax import lax
from jax.experimental import pallas as pl
from jax.experimental.pallas import tpu as pltpu
from jax.experimental.pallas import tpu_sc as plsc

N_NODES = 10000
N_EDGES = 320000
D = 128

NC = 2
NS = 16
NW = NC * NS
E_W = N_EDGES // NW
C = 96
K = 106
E_WP = K * C
NE_P = NW * E_WP
NP = 10240
RPT = NP // NS

CH = 200
KH = E_W // CH

_mesh = plsc.VectorSubcoreMesh(core_axis_name="c", subcore_axis_name="s")


def _fill(buf, rows, width, value):
    @pl.loop(0, rows)
    def _(r):
        for c in range(width // 16):
            buf.at[pl.ds(r, 1), pl.ds(c * 16, 16)][...] = jnp.full(
                (1, 16), value, jnp.float32)


def _zero_table(table, zbuf, zrows, base_r, rows):
    @pl.loop(0, rows // zrows)
    def _(i):
        pltpu.sync_copy(zbuf, table.at[pl.ds(base_r + i * zrows, zrows)])


@functools.partial(
    pl.kernel,
    out_type=jax.ShapeDtypeStruct((NC, NP, 16), jnp.float32),
    mesh=_mesh,
    scratch_types=[
        pltpu.VMEM((CH,), jnp.int32),
        pltpu.VMEM((CH, 16), jnp.float32),
        pltpu.VMEM((64, 16), jnp.float32),
        pltpu.VMEM_SHARED((NP, 16), jnp.float32),
    ],
)
def _sc_hist(dst_hbm, out_hbm, dst_v, ones_v, zbuf, table):
    cid = lax.axis_index("c")
    sid = lax.axis_index("s")
    wid = sid * NC + cid

    _fill(zbuf, 64, 16, 0.0)
    _fill(ones_v, CH, 16, 1.0)
    base_r = sid * RPT
    _zero_table(table, zbuf, 64, base_r, RPT)
    plsc.subcore_barrier()

    base_e = wid * E_W

    @pl.loop(0, KH)
    def _(j):
        pltpu.sync_copy(dst_hbm.at[pl.ds(base_e + j * CH, CH)], dst_v)
        pltpu.sync_copy(ones_v, table.at[dst_v], add=True)

    plsc.subcore_barrier()
    pltpu.sync_copy(table.at[pl.ds(base_r, RPT)],
                    out_hbm.at[cid, pl.ds(base_r, RPT)])


@functools.partial(
    pl.kernel,
    out_type=jax.ShapeDtypeStruct((NC, NP, D), jnp.float32),
    mesh=_mesh,
    scratch_types=[
        pltpu.VMEM((K, C), jnp.int32),
        pltpu.VMEM((1, C), jnp.int32),
        pltpu.VMEM((1, C), jnp.int32),
        pltpu.VMEM((1, C), jnp.int32),
        pltpu.VMEM((1, C), jnp.int32),
        pltpu.VMEM((C, D), jnp.float32),
        pltpu.VMEM((C, D), jnp.float32),
        pltpu.VMEM((8, D), jnp.float32),
        pltpu.VMEM_SHARED((NP, D), jnp.float32),
        pltpu.SemaphoreType.DMA,
        pltpu.SemaphoreType.DMA,
    ],
)
def _sc_scatter(hs_hbm, eidx_hbm, out_hbm, idx_all, src_0, src_1, dst_0,
                dst_1, rows_0, rows_1, zbuf, table, sem_0, sem_1):
    cid = lax.axis_index("c")
    sid = lax.axis_index("s")
    wid = sid * NC + cid
    src_v = (src_0, src_1)
    dst_v = (dst_0, dst_1)
    rows_v = (rows_0, rows_1)
    sems = (sem_0, sem_1)

    pltpu.sync_copy(eidx_hbm.at[wid], idx_all)

    _fill(zbuf, 8, D, 0.0)
    base_r = sid * RPT
    _zero_table(table, zbuf, 8, base_r, RPT)
    plsc.subcore_barrier()

    def unpack(chunk, b):
        for g in range(C // 16):
            sl = pl.ds(16 * g, 16)
            v = idx_all.at[pl.ds(chunk, 1), sl][...]
            src_v[b].at[pl.ds(0, 1), sl][...] = lax.shift_right_logical(v, 14)
            dst_v[b].at[pl.ds(0, 1), sl][...] = jnp.bitwise_and(v, 16383)

    def start_gather(b):
        pltpu.async_copy(hs_hbm.at[src_v[b].at[0]], rows_v[b], sems[b])

    def drain_gather(b):
        pltpu.make_async_copy(hs_hbm.at[pl.ds(0, C)], rows_v[b],
                              sems[b]).wait()

    for b in range(2):
        unpack(b, b)
        start_gather(b)

    @pl.loop(0, K // 2 - 1)
    def _(jj):
        j = jj * 2
        for b in range(2):
            drain_gather(b)
            pltpu.sync_copy(rows_v[b], table.at[dst_v[b].at[0]], add=True)
            unpack(j + b + 2, b)
            start_gather(b)

    for b in range(2):
        drain_gather(b)
        pltpu.sync_copy(rows_v[b], table.at[dst_v[b].at[0]], add=True)

    plsc.subcore_barrier()
    pltpu.sync_copy(table.at[pl.ds(base_r, RPT)],
                    out_hbm.at[cid, pl.ds(base_r, RPT)])


BLK = 2000
_GRID = N_NODES // BLK


def _row_spec(w):
    return pl.BlockSpec((BLK, w), lambda i: (i, 0))


def _full_spec(a, b):
    return pl.BlockSpec((a, b), lambda i: (0, 0))


def _mm_body(x_ref, w_ref, o_ref):
    o_ref[...] = jnp.dot(x_ref[...], w_ref[...],
                         preferred_element_type=jnp.float32)


def _mm(x, w):
    return pl.pallas_call(
        _mm_body,
        grid=(_GRID,),
        in_specs=[_row_spec(D), _full_spec(D, D)],
        out_specs=_row_spec(D),
        out_shape=jax.ShapeDtypeStruct((N_NODES, D), jnp.float32),
    )(x, w)


def _dinv(d0_ref, d1_ref):
    deg = d0_ref[...][:, :1] + d1_ref[...][:, :1] + 1.0
    return lax.rsqrt(deg)


def _scale_body(h_ref, d0_ref, d1_ref, o_ref):
    o_ref[...] = h_ref[...] * _dinv(d0_ref, d1_ref)


def _scale(h, d0, d1):
    return pl.pallas_call(
        _scale_body,
        grid=(_GRID,),
        in_specs=[_row_spec(D), _row_spec(16), _row_spec(16)],
        out_specs=_row_spec(D),
        out_shape=jax.ShapeDtypeStruct((N_NODES, D), jnp.float32),
    )(h, d0, d1)


def _mid_body(p0_ref, p1_ref, hs_ref, d0_ref, d1_ref, b_ref, w_ref, o_ref):
    dinv = _dinv(d0_ref, d1_ref)
    agg = p0_ref[...] + p1_ref[...]
    h = jnp.maximum(dinv * (agg + hs_ref[...]) + b_ref[...], 0.0)
    o_ref[...] = jnp.dot(h, w_ref[...],
                         preferred_element_type=jnp.float32) * dinv


def _mid(p0, p1, hs, d0, d1, b, w):
    return pl.pallas_call(
        _mid_body,
        grid=(_GRID,),
        in_specs=[_row_spec(D), _row_spec(D), _row_spec(D), _row_spec(16),
                  _row_spec(16), _full_spec(1, D), _full_spec(D, D)],
        out_specs=_row_spec(D),
        out_shape=jax.ShapeDtypeStruct((N_NODES, D), jnp.float32),
    )(p0, p1, hs, d0, d1, b, w)


def _out_body(p0_ref, p1_ref, hs_ref, d0_ref, d1_ref, b_ref, w_ref, b3_ref,
              o_ref):
    dinv = _dinv(d0_ref, d1_ref)
    agg = p0_ref[...] + p1_ref[...]
    h = jnp.maximum(dinv * (agg + hs_ref[...]) + b_ref[...], 0.0)
    o_ref[...] = jnp.dot(h, w_ref[...],
                         preferred_element_type=jnp.float32) + b3_ref[...]


def _out(p0, p1, hs, d0, d1, b, w3p, b3p):
    return pl.pallas_call(
        _out_body,
        grid=(_GRID,),
        in_specs=[_row_spec(D), _row_spec(D), _row_spec(D), _row_spec(16),
                  _row_spec(16), _full_spec(1, D), _full_spec(D, D),
                  _full_spec(1, D)],
        out_specs=_row_spec(D),
        out_shape=jax.ShapeDtypeStruct((N_NODES, D), jnp.float32),
    )(p0, p1, hs, d0, d1, b, w3p, b3p)


def kernel(x, edge_index, W1, b1, W2, b2, W3, b3):
    src = edge_index[0].astype(jnp.int32)
    dst = edge_index[1].astype(jnp.int32)
    pad_w = E_WP - E_W
    pad_src = jnp.broadcast_to(
        jnp.arange(pad_w, dtype=jnp.int32) % N_NODES, (NW, pad_w))
    pad_dst = jnp.broadcast_to(
        N_NODES + jnp.arange(pad_w, dtype=jnp.int32) % (NP - N_NODES),
        (NW, pad_w))
    src_p = jnp.concatenate([src.reshape(NW, E_W), pad_src], 1)
    dst_p = jnp.concatenate([dst.reshape(NW, E_W), pad_dst], 1)
    eidx = (jnp.left_shift(src_p, 14) | dst_p).reshape(NW, K, C)

    out_ch = W3.shape[1]
    w3p = jnp.zeros((D, D), jnp.float32).at[:, :out_ch].set(W3)
    b3p = jnp.zeros((1, D), jnp.float32).at[:, :out_ch].set(b3)

    deg_p = _sc_hist(dst)
    h_raw1 = _mm(x, W1)
    d0 = deg_p[0, :N_NODES]
    d1 = deg_p[1, :N_NODES]

    h1s = _scale(h_raw1, d0, d1)
    p1 = _sc_scatter(h1s, eidx)
    h2s = _mid(p1[0, :N_NODES], p1[1, :N_NODES], h1s, d0, d1,
               b1.reshape(1, D), W2)
    p2 = _sc_scatter(h2s, eidx)
    outp = _out(p2[0, :N_NODES], p2[1, :N_NODES], h2s, d0, d1,
                b2.reshape(1, D), w3p, b3p)
    return outp[:, :out_ch]
